# baseline (speedup 1.0000x reference)
import functools
import math

import jax
import jax.numpy as jnp
from jax import lax
from jax.experimental import pallas as pl
from jax.experimental.pallas import tpu as pltpu
from jax.experimental.pallas import tpu_sc as plsc

NUM_HEADS = 16
DIM = 64
MAX_MEMORIES = 32768
K = 32
Q = 32

_NEG = -3.0e38


_MM_CHUNK = 4096
_NMJ = MAX_MEMORIES // _MM_CHUNK


def _topk_body(q_ref, k_ref, scores_ref, idx_ref, s_scr, mi_scr):
    j = pl.program_id(1)

    @pl.when(j == 0)
    def _init_midx():
        c = jax.lax.broadcasted_iota(jnp.int32, (_NMJ, Q, _MM_CHUNK), 0)
        l = jax.lax.broadcasted_iota(jnp.int32, (_NMJ, Q, _MM_CHUNK), 2)
        mi_scr[...] = c * _MM_CHUNK + l

    kn = k_ref[0].astype(jnp.bfloat16)
    sims = jax.lax.dot_general(
        q_ref[0], kn, (((1,), (1,)), ((), ())),
        preferred_element_type=jnp.float32,
    ) * (1.0 / math.sqrt(DIM))
    s_scr[j] = sims

    @pl.when(j == _NMJ - 1)
    def _extract():
        eye = jnp.eye(Q, dtype=jnp.float32)

        def to_row(col):
            return jax.lax.dot_general(
                col, eye, (((0,), (0,)), ((), ())),
                preferred_element_type=jnp.float32,
                precision=jax.lax.Precision.HIGHEST)

        def extract(i, _):
            S = s_scr[...]
            MI = mi_scr[...]
            pm = jnp.max(S, axis=0)
            v = jnp.max(pm, axis=1, keepdims=True)
            amq = jnp.min(
                jnp.where(S == v[None, :, :], MI, MAX_MEMORIES), axis=0)
            am = jnp.min(amq, axis=1, keepdims=True)
            s_scr[...] = jnp.where(MI == am[None, :, :], _NEG, S)
            scores_ref[0, pl.ds(i, 1), :] = to_row(v)
            idx_ref[0, pl.ds(i, 1), :] = to_row(
                am.astype(jnp.float32)).astype(jnp.int32)
            return 0

        jax.lax.fori_loop(0, K, extract, 0)


def _find_topk(queries, key_memories):
    return pl.pallas_call(
        _topk_body,
        grid=(NUM_HEADS, _NMJ),
        in_specs=[
            pl.BlockSpec((1, Q, DIM), lambda h, j: (h, 0, 0)),
            pl.BlockSpec((1, _MM_CHUNK, DIM), lambda h, j: (h, j, 0)),
        ],
        out_specs=[
            pl.BlockSpec((1, K, Q), lambda h, j: (h, 0, 0)),
            pl.BlockSpec((1, K, Q), lambda h, j: (h, 0, 0)),
        ],
        out_shape=[
            jax.ShapeDtypeStruct((NUM_HEADS, K, Q), jnp.float32),
            jax.ShapeDtypeStruct((NUM_HEADS, K, Q), jnp.int32),
        ],
        scratch_shapes=[
            pltpu.VMEM((_NMJ, Q, _MM_CHUNK), jnp.float32),
            pltpu.VMEM((_NMJ, Q, _MM_CHUNK), jnp.int32),
        ],
    )(queries, key_memories)


def _make_sc_gather(V, D, B):
    info = plsc.get_sparse_core_info()
    NC, NS = info.num_cores, info.num_subcores
    NW = NC * NS
    assert B % (8 * NW) == 0 and D % info.num_lanes == 0
    b_per_w = B // NW
    mesh = plsc.VectorSubcoreMesh(core_axis_name="c", subcore_axis_name="s")

    @functools.partial(
        pl.kernel,
        mesh=mesh,
        compiler_params=pltpu.CompilerParams(use_tc_tiling_on_sc=False),
        out_type=jax.ShapeDtypeStruct((B, D), jnp.float32),
        scratch_types=[
            pltpu.VMEM((b_per_w,), jnp.int32),
            pltpu.VMEM((b_per_w, D), jnp.float32),
            pltpu.SemaphoreType.DMA,
        ],
    )
    def gather_kernel(table_hbm, idx_hbm, out_hbm, idx_v, rows_v, sem):
        wid = lax.axis_index("s") * NC + lax.axis_index("c")
        base = wid * b_per_w
        pltpu.sync_copy(idx_hbm.at[pl.ds(base, b_per_w)], idx_v)
        pltpu.async_copy(table_hbm.at[idx_v], rows_v, sem).wait()
        pltpu.sync_copy(rows_v, out_hbm.at[pl.ds(base, b_per_w)])

    return gather_kernel


def kernel(queries, key_memories, value_memories):
    qn = queries / (jnp.linalg.norm(queries, axis=-1, keepdims=True) + 1e-8)
    kn = key_memories / (jnp.linalg.norm(key_memories, axis=-1, keepdims=True) + 1e-8)
    scores_t, indices_t = _find_topk(qn.astype(jnp.bfloat16), kn)
    scores = scores_t.transpose(0, 2, 1)
    indices = indices_t.transpose(0, 2, 1)
    flat_values = value_memories.reshape(NUM_HEADS * MAX_MEMORIES, DIM)
    head_off = (jnp.arange(NUM_HEADS, dtype=jnp.int32) * MAX_MEMORIES)[:, None, None]
    flat_idx = (indices + head_off).reshape(NUM_HEADS * Q * K)
    gathered = _make_sc_gather(NUM_HEADS * MAX_MEMORIES, DIM, NUM_HEADS * Q * K)(
        flat_values, flat_idx)
    return (scores, indices, gathered.reshape(NUM_HEADS, Q, K, DIM))


# speedup vs baseline: 1.0216x; 1.0216x over previous
import functools
import math

import jax
import jax.numpy as jnp
from jax import lax
from jax.experimental import pallas as pl
from jax.experimental.pallas import tpu as pltpu
from jax.experimental.pallas import tpu_sc as plsc

NUM_HEADS = 16
DIM = 64
MAX_MEMORIES = 32768
K = 32
Q = 32

_NEG = -3.0e38


_MM_CHUNK = 4096
_NMJ = MAX_MEMORIES // _MM_CHUNK


def _topk_body(q_ref, k_ref, scores_ref, idx_ref, s_scr, mi_scr):
    j = pl.program_id(1)

    @pl.when(j == 0)
    def _init_midx():
        c = jax.lax.broadcasted_iota(jnp.int32, (_NMJ, Q, _MM_CHUNK), 0)
        l = jax.lax.broadcasted_iota(jnp.int32, (_NMJ, Q, _MM_CHUNK), 2)
        mi_scr[...] = c * _MM_CHUNK + l

    kn = k_ref[0]
    sims = jax.lax.dot_general(
        q_ref[0], kn, (((1,), (1,)), ((), ())),
        preferred_element_type=jnp.float32,
    ) * (1.0 / math.sqrt(DIM))
    s_scr[j] = sims

    @pl.when(j == _NMJ - 1)
    def _extract():
        eye = jnp.eye(Q, dtype=jnp.float32)

        def to_row(col):
            return jax.lax.dot_general(
                col, eye, (((0,), (0,)), ((), ())),
                preferred_element_type=jnp.float32,
                precision=jax.lax.Precision.HIGHEST)

        def extract(i, _):
            S = s_scr[...]
            MI = mi_scr[...]
            pm = jnp.max(S, axis=0)
            v = jnp.max(pm, axis=1, keepdims=True)
            amq = jnp.min(
                jnp.where(S == v[None, :, :], MI, MAX_MEMORIES), axis=0)
            am = jnp.min(amq, axis=1, keepdims=True)
            s_scr[...] = jnp.where(MI == am[None, :, :], _NEG, S)
            scores_ref[0, pl.ds(i, 1), :] = to_row(v)
            idx_ref[0, pl.ds(i, 1), :] = to_row(
                am.astype(jnp.float32)).astype(jnp.int32)
            return 0

        jax.lax.fori_loop(0, K, extract, 0)


def _find_topk(queries, key_memories):
    return pl.pallas_call(
        _topk_body,
        grid=(NUM_HEADS, _NMJ),
        in_specs=[
            pl.BlockSpec((1, Q, DIM), lambda h, j: (h, 0, 0)),
            pl.BlockSpec((1, _MM_CHUNK, DIM), lambda h, j: (h, j, 0)),
        ],
        out_specs=[
            pl.BlockSpec((1, K, Q), lambda h, j: (h, 0, 0)),
            pl.BlockSpec((1, K, Q), lambda h, j: (h, 0, 0)),
        ],
        out_shape=[
            jax.ShapeDtypeStruct((NUM_HEADS, K, Q), jnp.float32),
            jax.ShapeDtypeStruct((NUM_HEADS, K, Q), jnp.int32),
        ],
        scratch_shapes=[
            pltpu.VMEM((_NMJ, Q, _MM_CHUNK), jnp.float32),
            pltpu.VMEM((_NMJ, Q, _MM_CHUNK), jnp.int32),
        ],
        compiler_params=pltpu.CompilerParams(
            dimension_semantics=("parallel", "arbitrary")),
    )(queries, key_memories)


def _make_sc_gather(V, D, B):
    info = plsc.get_sparse_core_info()
    NC, NS = info.num_cores, info.num_subcores
    NW = NC * NS
    assert B % (8 * NW) == 0 and D % info.num_lanes == 0
    b_per_w = B // NW
    mesh = plsc.VectorSubcoreMesh(core_axis_name="c", subcore_axis_name="s")

    @functools.partial(
        pl.kernel,
        mesh=mesh,
        compiler_params=pltpu.CompilerParams(use_tc_tiling_on_sc=False),
        out_type=jax.ShapeDtypeStruct((B, D), jnp.float32),
        scratch_types=[
            pltpu.VMEM((b_per_w,), jnp.int32),
            pltpu.VMEM((b_per_w, D), jnp.float32),
            pltpu.SemaphoreType.DMA,
        ],
    )
    def gather_kernel(table_hbm, idx_hbm, out_hbm, idx_v, rows_v, sem):
        wid = lax.axis_index("s") * NC + lax.axis_index("c")
        base = wid * b_per_w
        pltpu.sync_copy(idx_hbm.at[pl.ds(base, b_per_w)], idx_v)
        pltpu.async_copy(table_hbm.at[idx_v], rows_v, sem).wait()
        pltpu.sync_copy(rows_v, out_hbm.at[pl.ds(base, b_per_w)])

    return gather_kernel


def kernel(queries, key_memories, value_memories):
    qn = queries / (jnp.linalg.norm(queries, axis=-1, keepdims=True) + 1e-8)
    kn = key_memories / (jnp.linalg.norm(key_memories, axis=-1, keepdims=True) + 1e-8)
    scores_t, indices_t = _find_topk(
        qn.astype(jnp.bfloat16), kn.astype(jnp.bfloat16))
    scores = scores_t.transpose(0, 2, 1)
    indices = indices_t.transpose(0, 2, 1)
    flat_values = value_memories.reshape(NUM_HEADS * MAX_MEMORIES, DIM)
    head_off = (jnp.arange(NUM_HEADS, dtype=jnp.int32) * MAX_MEMORIES)[:, None, None]
    flat_idx = (indices + head_off).reshape(NUM_HEADS * Q * K)
    gathered = _make_sc_gather(NUM_HEADS * MAX_MEMORIES, DIM, NUM_HEADS * Q * K)(
        flat_values, flat_idx)
    return (scores, indices, gathered.reshape(NUM_HEADS, Q, K, DIM))
